# baseline (speedup 1.0000x reference)
import functools

import jax
import jax.numpy as jnp
from jax import lax
from jax.experimental import pallas as pl
from jax.experimental.pallas import tpu as pltpu
from jax.experimental.pallas import tpu_sc as plsc

N_NODES = 10000
N_EDGES = 320000
D_IN = 128
D_HID = 64

NC = 2
NS = 16
NW = NC * NS
CHUNK = 128
NCHUNK = 80
EPW = CHUNK * NCHUNK
N_EDGES_PAD = NW * EPW
AGG_ROWS = N_NODES + 8
STRIPE = 1000
NSTRIPE_TILES = N_NODES // STRIPE


def _mm1_body(x_ref, w_ref, o_ref):
    o_ref[...] = jnp.dot(x_ref[...], w_ref[...],
                         preferred_element_type=jnp.float32)


def _epilogue_body(y_ref, p0_ref, p1_ref, b1_ref, w2_ref, b2_ref, o_ref):
    h = y_ref[...] + p0_ref[...] + p1_ref[...] + b1_ref[...]
    h = jnp.maximum(h, 0.0)
    o_ref[...] = jnp.dot(h, w2_ref[...],
                         preferred_element_type=jnp.float32) + b2_ref[...]


NBUF = 5


def _sc_segsum_body(ei_hbm, y_hbm, out_hbm,
                    si_v, di_v, rows, agg_sh, gsems, ssems):
    c = lax.axis_index("c")
    s = lax.axis_index("s")
    w = c * NS + s
    ebase = w * EPW

    pltpu.sync_copy(ei_hbm.at[0, pl.ds(ebase, EPW)], si_v)
    pltpu.sync_copy(ei_hbm.at[1, pl.ds(ebase, EPW)], di_v)

    zbase = s * STRIPE

    def _zrow(r, carry):
        for q in range(D_HID // 16):
            rows[0][r, pl.ds(q * 16, 16)] = jnp.zeros((16,), jnp.float32)
        return carry

    @pl.when(s < NSTRIPE_TILES)
    def _():
        lax.fori_loop(0, CHUNK, _zrow, None)
        for t in range(STRIPE // CHUNK):
            pltpu.sync_copy(rows[0], agg_sh.at[pl.ds(zbase + t * CHUNK, CHUNK)])
        rem = STRIPE % CHUNK
        if rem:
            pltpu.sync_copy(rows[0].at[pl.ds(0, rem)],
                            agg_sh.at[pl.ds(zbase + STRIPE - rem, rem)])
    plsc.subcore_barrier()

    def fire_g(b, i):
        off = pl.multiple_of(i * CHUNK, CHUNK)
        pltpu.async_copy(y_hbm.at[si_v.at[pl.ds(off, CHUNK)]], rows[b], gsems[b])

    def fire_s(b, i):
        off = pl.multiple_of(i * CHUNK, CHUNK)
        pltpu.async_copy(rows[b], agg_sh.at[di_v.at[pl.ds(off, CHUNK)]],
                         ssems[b], add=True)

    def wait_g(b):
        pltpu.make_async_copy(y_hbm.at[pl.ds(0, CHUNK)], rows[b], gsems[b]).wait()

    def wait_s(b):
        pltpu.make_async_copy(rows[b], agg_sh.at[di_v.at[pl.ds(0, CHUNK)]],
                              ssems[b]).wait()

    for b in range(NBUF):
        fire_g(b, b)

    def body(j, _):
        i = NBUF * j
        for b in range(NBUF):
            wait_g(b)
            fire_s(b, i + b)
            wait_s(b)
            fire_g(b, i + b + NBUF)
        return _

    lax.fori_loop(0, NCHUNK // NBUF - 1, body, None)
    for b in range(NBUF):
        wait_g(b)
        fire_s(b, NCHUNK - NBUF + b)
        wait_s(b)

    plsc.subcore_barrier()
    obase = c * N_NODES + s * STRIPE
    @pl.when(s < NSTRIPE_TILES)
    def _():
        pltpu.sync_copy(agg_sh.at[pl.ds(zbase, STRIPE)],
                        out_hbm.at[pl.ds(obase, STRIPE)])


@jax.jit
def kernel(x, edge_index, W1, b1, W2, b2):
    ei = edge_index.astype(jnp.int32)
    npad = N_EDGES_PAD - N_EDGES
    pad_blk = jnp.concatenate(
        [jnp.zeros((1, npad), jnp.int32),
         jnp.full((1, npad), N_NODES, jnp.int32)], axis=0)
    ei = jnp.concatenate([ei, pad_blk], axis=1)

    y = pl.pallas_call(
        _mm1_body,
        out_shape=jax.ShapeDtypeStruct((N_NODES, D_HID), jnp.float32),
    )(x, W1)

    sc_segsum = pl.kernel(
        _sc_segsum_body,
        out_type=jax.ShapeDtypeStruct((NC * N_NODES, D_HID), jnp.float32),
        mesh=plsc.VectorSubcoreMesh(core_axis_name="c", subcore_axis_name="s"),
        compiler_params=pltpu.CompilerParams(use_tc_tiling_on_sc=False),
        scratch_types=[
            pltpu.VMEM((EPW,), jnp.int32),
            pltpu.VMEM((EPW,), jnp.int32),
            [pltpu.VMEM((CHUNK, D_HID), jnp.float32)] * NBUF,
            pltpu.VMEM_SHARED((AGG_ROWS, D_HID), jnp.float32),
            [pltpu.SemaphoreType.DMA] * NBUF,
            [pltpu.SemaphoreType.DMA] * NBUF,
        ],
    )
    partials = sc_segsum(ei, y)

    out = pl.pallas_call(
        _epilogue_body,
        grid=(1,),
        in_specs=[
            pl.BlockSpec((N_NODES, D_HID), lambda i: (0, 0)),
            pl.BlockSpec((N_NODES, D_HID), lambda i: (0, 0)),
            pl.BlockSpec((N_NODES, D_HID), lambda i: (1, 0)),
            pl.BlockSpec((1, D_HID), lambda i: (0, 0)),
            pl.BlockSpec((D_HID, D_HID), lambda i: (0, 0)),
            pl.BlockSpec((1, D_HID), lambda i: (0, 0)),
        ],
        out_specs=pl.BlockSpec((N_NODES, D_HID), lambda i: (0, 0)),
        out_shape=jax.ShapeDtypeStruct((N_NODES, D_HID), jnp.float32),
    )(y, partials, partials, b1.reshape(1, D_HID), W2, b2.reshape(1, D_HID))
    return out


# speedup vs baseline: 2.8947x; 2.8947x over previous
import functools

import jax
import jax.numpy as jnp
from jax import lax
from jax.experimental import pallas as pl
from jax.experimental.pallas import tpu as pltpu
from jax.experimental.pallas import tpu_sc as plsc

N_NODES = 10000
N_EDGES = 320000
D_IN = 128
D_HID = 64

NC = 2
NS = 16
NW = NC * NS
CHUNK = 80
EPW = N_EDGES // NW
NCHUNK = EPW // CHUNK
AGG_ROWS = N_NODES
STRIPE = 1000
NSTRIPE_TILES = N_NODES // STRIPE


def _mm1_body(x_ref, w_ref, o_ref):
    o_ref[...] = jnp.dot(x_ref[...], w_ref[...],
                         preferred_element_type=jnp.float32)


def _epilogue_body(y_ref, p0_ref, p1_ref, b1_ref, w2_ref, b2_ref, o_ref):
    h = y_ref[...] + p0_ref[...] + p1_ref[...] + b1_ref[...]
    h = jnp.maximum(h, 0.0)
    o_ref[...] = jnp.dot(h, w2_ref[...],
                         preferred_element_type=jnp.float32) + b2_ref[...]


NBUF = 5


def _sc_segsum_body(ei_hbm, y_hbm, out_hbm,
                    si_v, di_v, rows, agg_sh, gsems, ssems):
    c = lax.axis_index("c")
    s = lax.axis_index("s")
    w = c * NS + s
    ebase = w * EPW

    pltpu.sync_copy(ei_hbm.at[0, pl.ds(ebase, EPW)], si_v)
    pltpu.sync_copy(ei_hbm.at[1, pl.ds(ebase, EPW)], di_v)

    zbase = s * STRIPE

    def _zrow(r, carry):
        for q in range(D_HID // 16):
            rows[0][r, pl.ds(q * 16, 16)] = jnp.zeros((16,), jnp.float32)
        return carry

    @pl.when(s < NSTRIPE_TILES)
    def _():
        lax.fori_loop(0, CHUNK, _zrow, None)
        for t in range(STRIPE // CHUNK):
            pltpu.sync_copy(rows[0], agg_sh.at[pl.ds(zbase + t * CHUNK, CHUNK)])
        rem = STRIPE % CHUNK
        if rem:
            pltpu.sync_copy(rows[0].at[pl.ds(0, rem)],
                            agg_sh.at[pl.ds(zbase + STRIPE - rem, rem)])
    plsc.subcore_barrier()

    def fire_g(b, i):
        off = pl.multiple_of(i * CHUNK, CHUNK)
        pltpu.async_copy(y_hbm.at[si_v.at[pl.ds(off, CHUNK)]], rows[b], gsems[b])

    def fire_s(b, i):
        off = pl.multiple_of(i * CHUNK, CHUNK)
        pltpu.async_copy(rows[b], agg_sh.at[di_v.at[pl.ds(off, CHUNK)]],
                         ssems[b], add=True)

    def wait_g(b):
        pltpu.make_async_copy(y_hbm.at[pl.ds(0, CHUNK)], rows[b], gsems[b]).wait()

    def wait_s(b):
        pltpu.make_async_copy(rows[b], agg_sh.at[di_v.at[pl.ds(0, CHUNK)]],
                              ssems[b]).wait()

    for b in range(NBUF):
        fire_g(b, b)

    def body(j, _):
        i = NBUF * j
        for b in range(NBUF):
            wait_g(b)
            fire_s(b, i + b)
            wait_s(b)
            fire_g(b, i + b + NBUF)
        return _

    lax.fori_loop(0, NCHUNK // NBUF - 1, body, None)
    for b in range(NBUF):
        wait_g(b)
        fire_s(b, NCHUNK - NBUF + b)
        wait_s(b)

    plsc.subcore_barrier()
    obase = c * N_NODES + s * STRIPE
    @pl.when(s < NSTRIPE_TILES)
    def _():
        pltpu.sync_copy(agg_sh.at[pl.ds(zbase, STRIPE)],
                        out_hbm.at[pl.ds(obase, STRIPE)])


@jax.jit
def kernel(x, edge_index, W1, b1, W2, b2):
    ei = edge_index.astype(jnp.int32)

    y = pl.pallas_call(
        _mm1_body,
        out_shape=jax.ShapeDtypeStruct((N_NODES, D_HID), jnp.float32),
    )(x, W1)

    sc_segsum = pl.kernel(
        _sc_segsum_body,
        out_type=jax.ShapeDtypeStruct((NC * N_NODES, D_HID), jnp.float32),
        mesh=plsc.VectorSubcoreMesh(core_axis_name="c", subcore_axis_name="s"),
        compiler_params=pltpu.CompilerParams(use_tc_tiling_on_sc=False),
        scratch_types=[
            pltpu.VMEM((EPW,), jnp.int32),
            pltpu.VMEM((EPW,), jnp.int32),
            [pltpu.VMEM((CHUNK, D_HID), jnp.float32)] * NBUF,
            pltpu.VMEM_SHARED((AGG_ROWS, D_HID), jnp.float32),
            [pltpu.SemaphoreType.DMA] * NBUF,
            [pltpu.SemaphoreType.DMA] * NBUF,
        ],
    )
    partials = sc_segsum(ei, y)

    out = pl.pallas_call(
        _epilogue_body,
        grid=(1,),
        in_specs=[
            pl.BlockSpec((N_NODES, D_HID), lambda i: (0, 0)),
            pl.BlockSpec((N_NODES, D_HID), lambda i: (0, 0)),
            pl.BlockSpec((N_NODES, D_HID), lambda i: (1, 0)),
            pl.BlockSpec((1, D_HID), lambda i: (0, 0)),
            pl.BlockSpec((D_HID, D_HID), lambda i: (0, 0)),
            pl.BlockSpec((1, D_HID), lambda i: (0, 0)),
        ],
        out_specs=pl.BlockSpec((N_NODES, D_HID), lambda i: (0, 0)),
        out_shape=jax.ShapeDtypeStruct((N_NODES, D_HID), jnp.float32),
    )(y, partials, partials, b1.reshape(1, D_HID), W2, b2.reshape(1, D_HID))
    return out


# speedup vs baseline: 2.8964x; 1.0006x over previous
import jax
import jax.numpy as jnp
from jax import lax
from jax.experimental import pallas as pl
from jax.experimental.pallas import tpu as pltpu
from jax.experimental.pallas import tpu_sc as plsc

N_NODES = 10000
N_EDGES = 320000
D_IN = 128
D_HID = 64

NC = 2
NS = 16
NW = NC * NS
CHUNK = 80
EPW = N_EDGES // NW
NCHUNK = EPW // CHUNK
AGG_ROWS = N_NODES
STRIPE = 1000
NSTRIPE_TILES = N_NODES // STRIPE


def _mm1_body(x_ref, w_ref, o_ref):
    o_ref[...] = jnp.dot(x_ref[...], w_ref[...],
                         preferred_element_type=jnp.float32)


def _epilogue_body(y_ref, p0_ref, p1_ref, b1_ref, w2_ref, b2_ref, o_ref):
    h = y_ref[...] + p0_ref[...] + p1_ref[...] + b1_ref[...]
    h = jnp.maximum(h, 0.0)
    o_ref[...] = jnp.dot(h, w2_ref[...],
                         preferred_element_type=jnp.float32) + b2_ref[...]


NBUF = 5


def _sc_segsum_body(ei_hbm, y_hbm, out_hbm,
                    si_v, di_v, rows, agg_sh, gsems, ssems):
    c = lax.axis_index("c")
    s = lax.axis_index("s")
    w = c * NS + s
    ebase = w * EPW

    pltpu.sync_copy(ei_hbm.at[0, pl.ds(ebase, EPW)], si_v)
    pltpu.sync_copy(ei_hbm.at[1, pl.ds(ebase, EPW)], di_v)

    zbase = s * STRIPE

    def _zrow(r, carry):
        for q in range(D_HID // 16):
            rows[0][r, pl.ds(q * 16, 16)] = jnp.zeros((16,), jnp.float32)
        return carry

    @pl.when(s < NSTRIPE_TILES)
    def _():
        lax.fori_loop(0, CHUNK, _zrow, None)
        for t in range(STRIPE // CHUNK):
            pltpu.sync_copy(rows[0], agg_sh.at[pl.ds(zbase + t * CHUNK, CHUNK)])
        rem = STRIPE % CHUNK
        if rem:
            pltpu.sync_copy(rows[0].at[pl.ds(0, rem)],
                            agg_sh.at[pl.ds(zbase + STRIPE - rem, rem)])
    plsc.subcore_barrier()

    def fire_g(b, i):
        off = pl.multiple_of(i * CHUNK, CHUNK)
        pltpu.async_copy(y_hbm.at[si_v.at[pl.ds(off, CHUNK)]], rows[b], gsems[b])

    def fire_s(b, i):
        off = pl.multiple_of(i * CHUNK, CHUNK)
        pltpu.async_copy(rows[b], agg_sh.at[di_v.at[pl.ds(off, CHUNK)]],
                         ssems[b], add=True)

    def wait_g(b):
        pltpu.make_async_copy(y_hbm.at[pl.ds(0, CHUNK)], rows[b], gsems[b]).wait()

    def wait_s(b):
        pltpu.make_async_copy(rows[b], agg_sh.at[di_v.at[pl.ds(0, CHUNK)]],
                              ssems[b]).wait()

    for b in range(NBUF):
        fire_g(b, b)

    def body(j, _):
        i = NBUF * j
        for b in range(NBUF):
            wait_g(b)
            fire_s(b, i + b)
            wait_s(b)
            fire_g(b, i + b + NBUF)
        return _

    lax.fori_loop(0, NCHUNK // NBUF - 1, body, None)
    for b in range(NBUF):
        wait_g(b)
        fire_s(b, NCHUNK - NBUF + b)
        wait_s(b)

    plsc.subcore_barrier()
    obase = c * N_NODES + s * STRIPE
    @pl.when(s < NSTRIPE_TILES)
    def _():
        pltpu.sync_copy(agg_sh.at[pl.ds(zbase, STRIPE)],
                        out_hbm.at[pl.ds(obase, STRIPE)])


@jax.jit
def kernel(x, edge_index, W1, b1, W2, b2):
    ei = edge_index.astype(jnp.int32)

    y = pl.pallas_call(
        _mm1_body,
        out_shape=jax.ShapeDtypeStruct((N_NODES, D_HID), jnp.float32),
    )(x, W1)

    sc_segsum = pl.kernel(
        _sc_segsum_body,
        out_type=jax.ShapeDtypeStruct((NC * N_NODES, D_HID), jnp.float32),
        mesh=plsc.VectorSubcoreMesh(core_axis_name="c", subcore_axis_name="s"),
        compiler_params=pltpu.CompilerParams(use_tc_tiling_on_sc=False),
        scratch_types=[
            pltpu.VMEM((EPW,), jnp.int32),
            pltpu.VMEM((EPW,), jnp.int32),
            [pltpu.VMEM((CHUNK, D_HID), jnp.float32)] * NBUF,
            pltpu.VMEM_SHARED((AGG_ROWS, D_HID), jnp.float32),
            [pltpu.SemaphoreType.DMA] * NBUF,
            [pltpu.SemaphoreType.DMA] * NBUF,
        ],
    )
    partials = sc_segsum(ei, y)

    out = pl.pallas_call(
        _epilogue_body,
        grid=(1,),
        in_specs=[
            pl.BlockSpec((N_NODES, D_HID), lambda i: (0, 0)),
            pl.BlockSpec((N_NODES, D_HID), lambda i: (0, 0)),
            pl.BlockSpec((N_NODES, D_HID), lambda i: (1, 0)),
            pl.BlockSpec((1, D_HID), lambda i: (0, 0)),
            pl.BlockSpec((D_HID, D_HID), lambda i: (0, 0)),
            pl.BlockSpec((1, D_HID), lambda i: (0, 0)),
        ],
        out_specs=pl.BlockSpec((N_NODES, D_HID), lambda i: (0, 0)),
        out_shape=jax.ShapeDtypeStruct((N_NODES, D_HID), jnp.float32),
    )(y, partials, partials, b1.reshape(1, D_HID), W2, b2.reshape(1, D_HID))
    return out
